# baseline (speedup 1.0000x reference)
import jax
import jax.numpy as jnp
from jax.experimental import pallas as pl
from jax.experimental.pallas import tpu as pltpu
from jax.experimental.pallas import tpu_sc as plsc

E = 800000
K = 64
CH = 128
L = 16

_mesh = plsc.VectorSubcoreMesh(core_axis_name="core", subcore_axis_name="subcore")


def _sc_body(a_v, b_v, c_v, d_v, o_v):
    @pl.loop(0, CH, step=L)
    def _(c1):
        def kstep(k, acc):
            sl = (k, pl.ds(c1, L))
            return acc + a_v[sl] * b_v[sl] + c_v[sl] * d_v[sl]
        acc = jax.lax.fori_loop(0, K, kstep, jnp.zeros((L,), jnp.float32))
        o_v[pl.ds(c1, L)] = acc


def kernel(gu, gi, gut, git):
    @pl.kernel(
        out_type=jax.ShapeDtypeStruct((E,), jnp.float32),
        mesh=_mesh,
    )
    def sc_kernel(a_hbm, b_hbm, c_hbm, d_hbm, o_hbm):
        in_spec = pl.BlockSpec((K, CH), lambda i: (0, i))
        pltpu.emit_pipeline(
            _sc_body,
            grid=(E // CH,),
            in_specs=[in_spec, in_spec, in_spec, in_spec],
            out_specs=[pl.BlockSpec((CH,), lambda i: (i,))],
            core_axis_name=("core", "subcore"),
            dimension_semantics=(pltpu.PARALLEL,),
        )(a_hbm, b_hbm, c_hbm, d_hbm, o_hbm)

    return sc_kernel(gu.T, gi.T, gut.T, git.T)


# speedup vs baseline: 1.3820x; 1.3820x over previous
import jax
import jax.numpy as jnp
from jax.experimental import pallas as pl
from jax.experimental.pallas import tpu as pltpu
from jax.experimental.pallas import tpu_sc as plsc

E = 800000
K = 64
BW = 8192
CH = 128
L = 16
E_SC = 311296
E_TC = E - E_SC

_mesh = plsc.VectorSubcoreMesh(core_axis_name="core", subcore_axis_name="subcore")


def _tc_body(gu_ref, gi_ref, gut_ref, git_ref, out_ref):
    p = gu_ref[...] * gi_ref[...] + gut_ref[...] * git_ref[...]
    out_ref[...] = jnp.sum(p, axis=0)


def _sc_body(a_v, b_v, c_v, d_v, o_v):
    @pl.loop(0, CH, step=L)
    def _(c1):
        def kstep(k, acc):
            sl = (k, pl.ds(c1, L))
            return acc + a_v[sl] * b_v[sl] + c_v[sl] * d_v[sl]
        acc = jax.lax.fori_loop(0, K, kstep, jnp.zeros((L,), jnp.float32))
        o_v[pl.ds(c1, L)] = acc


def kernel(gu, gi, gut, git):
    gut_t = gu.T
    git_t = gi.T
    gutt_t = gut.T
    gitt_t = git.T

    @pl.kernel(
        out_type=jax.ShapeDtypeStruct((E_SC,), jnp.float32),
        mesh=_mesh,
    )
    def sc_kernel(a_hbm, b_hbm, c_hbm, d_hbm, o_hbm):
        off = E_TC // CH
        in_spec = pl.BlockSpec((K, CH), lambda i: (0, i + off))
        pltpu.emit_pipeline(
            _sc_body,
            grid=(E_SC // CH,),
            in_specs=[in_spec, in_spec, in_spec, in_spec],
            out_specs=[pl.BlockSpec((CH,), lambda i: (i,))],
            core_axis_name=("core", "subcore"),
            dimension_semantics=(pltpu.PARALLEL,),
        )(a_hbm, b_hbm, c_hbm, d_hbm, o_hbm)

    sc_out = sc_kernel(gut_t, git_t, gutt_t, gitt_t)

    in_spec = pl.BlockSpec((K, BW), lambda i: (0, i))
    tc_out = pl.pallas_call(
        _tc_body,
        grid=((E_TC + BW - 1) // BW,),
        in_specs=[in_spec, in_spec, in_spec, in_spec],
        out_specs=pl.BlockSpec((BW,), lambda i: (i,)),
        out_shape=jax.ShapeDtypeStruct((E_TC,), jnp.float32),
    )(gut_t, git_t, gutt_t, gitt_t)

    return jnp.concatenate([tc_out, sc_out])


# speedup vs baseline: 1.3883x; 1.0045x over previous
import jax
import jax.numpy as jnp
from jax.experimental import pallas as pl
from jax.experimental.pallas import tpu as pltpu
from jax.experimental.pallas import tpu_sc as plsc

E = 800000
K = 64
BW = 8192
CH = 128
L = 16
E_SC = 249600
E_TC = E - E_SC

_mesh = plsc.VectorSubcoreMesh(core_axis_name="core", subcore_axis_name="subcore")


def _tc_body(gu_ref, gi_ref, gut_ref, git_ref, out_ref):
    p = gu_ref[...] * gi_ref[...] + gut_ref[...] * git_ref[...]
    out_ref[...] = jnp.sum(p, axis=0)


def _sc_body(a_v, b_v, c_v, d_v, o_v):
    @pl.loop(0, CH, step=L)
    def _(c1):
        def kstep(k, acc):
            sl = (k, pl.ds(c1, L))
            return acc + a_v[sl] * b_v[sl] + c_v[sl] * d_v[sl]
        acc = jax.lax.fori_loop(0, K, kstep, jnp.zeros((L,), jnp.float32))
        o_v[pl.ds(c1, L)] = acc


def kernel(gu, gi, gut, git):
    gut_t = gu.T
    git_t = gi.T
    gutt_t = gut.T
    gitt_t = git.T

    @pl.kernel(
        out_type=jax.ShapeDtypeStruct((E_SC,), jnp.float32),
        mesh=_mesh,
    )
    def sc_kernel(a_hbm, b_hbm, c_hbm, d_hbm, o_hbm):
        off = E_TC // CH
        in_spec = pl.BlockSpec((K, CH), lambda i: (0, i + off))
        pltpu.emit_pipeline(
            _sc_body,
            grid=(E_SC // CH,),
            in_specs=[in_spec, in_spec, in_spec, in_spec],
            out_specs=[pl.BlockSpec((CH,), lambda i: (i,))],
            core_axis_name=("core", "subcore"),
            dimension_semantics=(pltpu.PARALLEL,),
        )(a_hbm, b_hbm, c_hbm, d_hbm, o_hbm)

    sc_out = sc_kernel(gut_t, git_t, gutt_t, gitt_t)

    in_spec = pl.BlockSpec((K, BW), lambda i: (0, i))
    tc_out = pl.pallas_call(
        _tc_body,
        grid=((E_TC + BW - 1) // BW,),
        in_specs=[in_spec, in_spec, in_spec, in_spec],
        out_specs=pl.BlockSpec((BW,), lambda i: (i,)),
        out_shape=jax.ShapeDtypeStruct((E_TC,), jnp.float32),
    )(gut_t, git_t, gutt_t, gitt_t)

    return jnp.concatenate([tc_out, sc_out])


# speedup vs baseline: 1.3980x; 1.0070x over previous
import jax
import jax.numpy as jnp
from jax.experimental import pallas as pl
from jax.experimental.pallas import tpu as pltpu
from jax.experimental.pallas import tpu_sc as plsc

E = 800000
K = 64
BW = 8192
CH = 1024
KR = 8
L = 16
E_SC = 262144
E_TC = E - E_SC

_mesh = plsc.VectorSubcoreMesh(core_axis_name="core", subcore_axis_name="subcore")


def _tc_body(gu_ref, gi_ref, gut_ref, git_ref, out_ref):
    p = gu_ref[...] * gi_ref[...] + gut_ref[...] * git_ref[...]
    out_ref[...] = jnp.sum(p, axis=0)


def _sc_body(idx, a_v, b_v, c_v, d_v, o_v):
    j = idx[1]

    @pl.loop(0, CH, step=L)
    def _(c1):
        def kstep(k, acc):
            sl = (k, pl.ds(c1, L))
            return acc + a_v[sl] * b_v[sl] + c_v[sl] * d_v[sl]
        acc = jax.lax.fori_loop(0, KR, kstep, jnp.zeros((L,), jnp.float32))

        @pl.when(j == 0)
        def _():
            o_v[pl.ds(c1, L)] = acc

        @pl.when(j != 0)
        def _():
            o_v[pl.ds(c1, L)] = o_v[pl.ds(c1, L)] + acc


def kernel(gu, gi, gut, git):
    gut_t = gu.T
    git_t = gi.T
    gutt_t = gut.T
    gitt_t = git.T

    @pl.kernel(
        out_type=jax.ShapeDtypeStruct((E_SC,), jnp.float32),
        mesh=_mesh,
    )
    def sc_kernel(a_hbm, b_hbm, c_hbm, d_hbm, o_hbm):
        in_spec = pl.BlockSpec((KR, CH), lambda i, j: (j, i))
        pltpu.emit_pipeline(
            _sc_body,
            grid=(E_SC // CH, K // KR),
            in_specs=[in_spec, in_spec, in_spec, in_spec],
            out_specs=[pl.BlockSpec((CH,), lambda i, j: (i,))],
            core_axis_name=("core", "subcore"),
            dimension_semantics=(pltpu.PARALLEL, pltpu.ARBITRARY),
            _explicit_indices=True,
        )(a_hbm, b_hbm, c_hbm, d_hbm, o_hbm)

    sc_out = sc_kernel(gut_t, git_t, gutt_t, gitt_t)

    tc_off = E_SC // BW
    in_spec = pl.BlockSpec((K, BW), lambda i: (0, i + tc_off))
    tc_out = pl.pallas_call(
        _tc_body,
        grid=((E_TC + BW - 1) // BW,),
        in_specs=[in_spec, in_spec, in_spec, in_spec],
        out_specs=pl.BlockSpec((BW,), lambda i: (i,)),
        out_shape=jax.ShapeDtypeStruct((E_TC,), jnp.float32),
    )(gut_t, git_t, gutt_t, gitt_t)

    return jnp.concatenate([sc_out, tc_out])


# speedup vs baseline: 1.4216x; 1.0169x over previous
import jax
import jax.numpy as jnp
from jax.experimental import pallas as pl
from jax.experimental.pallas import tpu as pltpu
from jax.experimental.pallas import tpu_sc as plsc

E = 800000
K = 64
BW = 8192
CH = 1024
KR = 8
L = 16
E_SC = 131072
E_TC = E - E_SC

_mesh = plsc.VectorSubcoreMesh(core_axis_name="core", subcore_axis_name="subcore")


def _tc_body(gu_ref, gi_ref, gut_ref, git_ref, out_ref):
    p = gu_ref[...] * gi_ref[...] + gut_ref[...] * git_ref[...]
    out_ref[...] = jnp.sum(p, axis=0)


def _sc_body(idx, a_v, b_v, c_v, d_v, o_v):
    j = idx[1]

    @pl.loop(0, CH, step=L)
    def _(c1):
        def kstep(k, acc):
            sl = (k, pl.ds(c1, L))
            return acc + a_v[sl] * b_v[sl] + c_v[sl] * d_v[sl]
        acc = jax.lax.fori_loop(0, KR, kstep, jnp.zeros((L,), jnp.float32))

        @pl.when(j == 0)
        def _():
            o_v[pl.ds(c1, L)] = acc

        @pl.when(j != 0)
        def _():
            o_v[pl.ds(c1, L)] = o_v[pl.ds(c1, L)] + acc


def kernel(gu, gi, gut, git):
    gut_t = gu.T
    git_t = gi.T
    gutt_t = gut.T
    gitt_t = git.T

    @pl.kernel(
        out_type=jax.ShapeDtypeStruct((E_SC,), jnp.float32),
        mesh=_mesh,
    )
    def sc_kernel(a_hbm, b_hbm, c_hbm, d_hbm, o_hbm):
        in_spec = pl.BlockSpec((KR, CH), lambda i, j: (j, i))
        pltpu.emit_pipeline(
            _sc_body,
            grid=(E_SC // CH, K // KR),
            in_specs=[in_spec, in_spec, in_spec, in_spec],
            out_specs=[pl.BlockSpec((CH,), lambda i, j: (i,))],
            core_axis_name=("core", "subcore"),
            dimension_semantics=(pltpu.PARALLEL, pltpu.ARBITRARY),
            _explicit_indices=True,
        )(a_hbm, b_hbm, c_hbm, d_hbm, o_hbm)

    sc_out = sc_kernel(gut_t, git_t, gutt_t, gitt_t)

    tc_off = E_SC // BW
    in_spec = pl.BlockSpec((K, BW), lambda i: (0, i + tc_off))
    tc_out = pl.pallas_call(
        _tc_body,
        grid=((E_TC + BW - 1) // BW,),
        in_specs=[in_spec, in_spec, in_spec, in_spec],
        out_specs=pl.BlockSpec((BW,), lambda i: (i,)),
        out_shape=jax.ShapeDtypeStruct((E_TC,), jnp.float32),
    )(gut_t, git_t, gutt_t, gitt_t)

    return jnp.concatenate([sc_out, tc_out])


# speedup vs baseline: 1.4288x; 1.0051x over previous
import jax
import jax.numpy as jnp
from jax.experimental import pallas as pl
from jax.experimental.pallas import tpu as pltpu
from jax.experimental.pallas import tpu_sc as plsc

E = 800000
K = 64
BW = 8192
CH = 1024
KR = 8
L = 16
E_SC = 65536
E_TC = E - E_SC

_mesh = plsc.VectorSubcoreMesh(core_axis_name="core", subcore_axis_name="subcore")


def _tc_body(gu_ref, gi_ref, gut_ref, git_ref, out_ref):
    p = gu_ref[...] * gi_ref[...] + gut_ref[...] * git_ref[...]
    out_ref[...] = jnp.sum(p, axis=0)


def _sc_body(idx, a_v, b_v, c_v, d_v, o_v):
    j = idx[1]

    @pl.loop(0, CH, step=L)
    def _(c1):
        def kstep(k, acc):
            sl = (k, pl.ds(c1, L))
            return acc + a_v[sl] * b_v[sl] + c_v[sl] * d_v[sl]
        acc = jax.lax.fori_loop(0, KR, kstep, jnp.zeros((L,), jnp.float32))

        @pl.when(j == 0)
        def _():
            o_v[pl.ds(c1, L)] = acc

        @pl.when(j != 0)
        def _():
            o_v[pl.ds(c1, L)] = o_v[pl.ds(c1, L)] + acc


def kernel(gu, gi, gut, git):
    gut_t = gu.T
    git_t = gi.T
    gutt_t = gut.T
    gitt_t = git.T

    @pl.kernel(
        out_type=jax.ShapeDtypeStruct((E_SC,), jnp.float32),
        mesh=_mesh,
    )
    def sc_kernel(a_hbm, b_hbm, c_hbm, d_hbm, o_hbm):
        in_spec = pl.BlockSpec((KR, CH), lambda i, j: (j, i))
        pltpu.emit_pipeline(
            _sc_body,
            grid=(E_SC // CH, K // KR),
            in_specs=[in_spec, in_spec, in_spec, in_spec],
            out_specs=[pl.BlockSpec((CH,), lambda i, j: (i,))],
            core_axis_name=("core", "subcore"),
            dimension_semantics=(pltpu.PARALLEL, pltpu.ARBITRARY),
            _explicit_indices=True,
        )(a_hbm, b_hbm, c_hbm, d_hbm, o_hbm)

    sc_out = sc_kernel(gut_t, git_t, gutt_t, gitt_t)

    tc_off = E_SC // BW
    in_spec = pl.BlockSpec((K, BW), lambda i: (0, i + tc_off))
    tc_out = pl.pallas_call(
        _tc_body,
        grid=((E_TC + BW - 1) // BW,),
        in_specs=[in_spec, in_spec, in_spec, in_spec],
        out_specs=pl.BlockSpec((BW,), lambda i: (i,)),
        out_shape=jax.ShapeDtypeStruct((E_TC,), jnp.float32),
    )(gut_t, git_t, gutt_t, gitt_t)

    return jnp.concatenate([sc_out, tc_out])


# speedup vs baseline: 1.4367x; 1.0055x over previous
import jax
import jax.numpy as jnp
from jax.experimental import pallas as pl
from jax.experimental.pallas import tpu as pltpu
from jax.experimental.pallas import tpu_sc as plsc

E = 800000
K = 64
BW = 8192
CH = 1024
KR = 8
L = 16
E_SC = 32768
E_TC = E - E_SC

_mesh = plsc.VectorSubcoreMesh(core_axis_name="core", subcore_axis_name="subcore")


def _tc_body(gu_ref, gi_ref, gut_ref, git_ref, out_ref):
    p = gu_ref[...] * gi_ref[...] + gut_ref[...] * git_ref[...]
    out_ref[...] = jnp.sum(p, axis=0)


def _sc_body(idx, a_v, b_v, c_v, d_v, o_v):
    j = idx[1]

    @pl.loop(0, CH, step=L)
    def _(c1):
        def kstep(k, acc):
            sl = (k, pl.ds(c1, L))
            return acc + a_v[sl] * b_v[sl] + c_v[sl] * d_v[sl]
        acc = jax.lax.fori_loop(0, KR, kstep, jnp.zeros((L,), jnp.float32))

        @pl.when(j == 0)
        def _():
            o_v[pl.ds(c1, L)] = acc

        @pl.when(j != 0)
        def _():
            o_v[pl.ds(c1, L)] = o_v[pl.ds(c1, L)] + acc


def kernel(gu, gi, gut, git):
    gut_t = gu.T
    git_t = gi.T
    gutt_t = gut.T
    gitt_t = git.T

    @pl.kernel(
        out_type=jax.ShapeDtypeStruct((E_SC,), jnp.float32),
        mesh=_mesh,
    )
    def sc_kernel(a_hbm, b_hbm, c_hbm, d_hbm, o_hbm):
        in_spec = pl.BlockSpec((KR, CH), lambda i, j: (j, i))
        pltpu.emit_pipeline(
            _sc_body,
            grid=(E_SC // CH, K // KR),
            in_specs=[in_spec, in_spec, in_spec, in_spec],
            out_specs=[pl.BlockSpec((CH,), lambda i, j: (i,))],
            core_axis_name=("core", "subcore"),
            dimension_semantics=(pltpu.PARALLEL, pltpu.ARBITRARY),
            _explicit_indices=True,
        )(a_hbm, b_hbm, c_hbm, d_hbm, o_hbm)

    sc_out = sc_kernel(gut_t, git_t, gutt_t, gitt_t)

    tc_off = E_SC // BW
    in_spec = pl.BlockSpec((K, BW), lambda i: (0, i + tc_off))
    tc_out = pl.pallas_call(
        _tc_body,
        grid=((E_TC + BW - 1) // BW,),
        in_specs=[in_spec, in_spec, in_spec, in_spec],
        out_specs=pl.BlockSpec((BW,), lambda i: (i,)),
        out_shape=jax.ShapeDtypeStruct((E_TC,), jnp.float32),
    )(gut_t, git_t, gutt_t, gitt_t)

    return jnp.concatenate([sc_out, tc_out])
